# Initial kernel scaffold; baseline (speedup 1.0000x reference)
import jax
import jax.numpy as jnp
from jax.experimental import pallas as pl


def kernel(pred_logits, pred_boxes, target_sizes):
    raise NotImplementedError("write your pallas kernel here")


# speedup vs baseline: 1.9260x; 1.9260x over previous
import functools

import jax
import jax.numpy as jnp
from jax import lax
from jax.experimental import pallas as pl
from jax.experimental.pallas import tpu as pltpu

_B, _N, _C = 16, 20000, 91
_NC = _N * _C
_S = 1024
_MR, _MC = 14, 128
_M = _MR * _MC
_NCP = _M * _S
_K = 100
_KP = 128
_BIG = 1 << 30
_NEG = -jnp.inf


def _probs(x):
    return jnp.where(x == _NEG, jnp.float32(-1.0), jax.nn.sigmoid(x))


def _topk_body(x_hbm, pb_hbm, scale_ref, scores_ref, labels_ref, boxes_ref,
               xv, cm, am, val_s, idx_s, bx_s, sem_x, sem_b):
    b = pl.program_id(0)
    pltpu.make_async_copy(x_hbm.at[b], xv, sem_x).start()
    pltpu.make_async_copy(x_hbm.at[b], xv, sem_x).wait()

    for r in range(_MR):
        blk = xv[r * _MC:(r + 1) * _MC, :]
        p = _probs(blk)
        m = jnp.max(p, axis=1)
        ii = lax.broadcasted_iota(jnp.int32, (_MC, _S), 1)
        a = jnp.min(jnp.where(p == m[:, None], ii, _BIG), axis=1)
        cm[r, :] = m
        am[r, :] = a

    lin = (lax.broadcasted_iota(jnp.int32, (_MR, _MC), 0) * _MC
           + lax.broadcasted_iota(jnp.int32, (_MR, _MC), 1))
    li = lax.broadcasted_iota(jnp.int32, (1, _S), 1)

    def body(k, carry):
        cmv = cm[:, :]
        gm = jnp.max(cmv)
        j = jnp.min(jnp.where(cmv == gm, lin, _BIG))
        off = jnp.min(jnp.where(lin == j, am[:, :], _BIG))
        gidx = j * _S + off
        val_s[pl.ds(k, 1), :] = jnp.reshape(gm, (1, 1))
        idx_s[pl.ds(k, 1), :] = jnp.reshape(gidx, (1, 1))
        bi = jnp.floor((gidx.astype(jnp.float32) + 0.5) *
                       jnp.float32(1.0 / _C)).astype(jnp.int32)
        pltpu.make_async_copy(pb_hbm.at[b, pl.ds(bi, 1), :],
                              bx_s.at[pl.ds(k, 1), :], sem_b).start()
        row = xv[pl.ds(j, 1), :]
        nrow = jnp.where(li == off, _NEG, row)
        xv[pl.ds(j, 1), :] = nrow
        pr = _probs(nrow)
        rm = jnp.max(pr)
        ra = jnp.min(jnp.where(pr == rm, li, _BIG))
        cm[:, :] = jnp.where(lin == j, rm, cmv)
        am[:, :] = jnp.where(lin == j, ra, am[:, :])
        return carry

    lax.fori_loop(0, _K, body, 0)

    pltpu.make_async_copy(pb_hbm.at[b, pl.ds(0, _K), :],
                          bx_s.at[pl.ds(0, _K), :], sem_b).wait()

    scores_ref[0] = val_s[:, :]
    rows = lax.broadcasted_iota(jnp.int32, (_KP, 1), 0)
    g = jnp.where(rows < _K, idx_s[:, :], 0)
    q = jnp.floor((g.astype(jnp.float32) + 0.5) *
                  jnp.float32(1.0 / _C)).astype(jnp.int32)
    labels_ref[0] = g - q * _C

    bx = bx_s[:, :]
    cx = bx[:, 0:1]
    cy = bx[:, 1:2]
    w2 = 0.5 * bx[:, 2:3]
    h2 = 0.5 * bx[:, 3:4]
    xy = jnp.concatenate([cx - w2, cy - h2, cx + w2, cy + h2], axis=1)
    boxes_ref[0] = xy * scale_ref[0]


@jax.jit
def kernel(pred_logits, pred_boxes, target_sizes):
    B, N, C = pred_logits.shape
    flat = pred_logits.reshape(B, N * C)
    xp = jnp.pad(flat, ((0, 0), (0, _NCP - N * C)),
                 constant_values=_NEG).reshape(B, _M, _S)
    ts = target_sizes.astype(jnp.float32)
    h = ts[:, 0:1]
    w = ts[:, 1:2]
    scale = jnp.concatenate([w, h, w, h], axis=1).reshape(B, 1, 4)

    scores3, labels3, boxes3 = pl.pallas_call(
        _topk_body,
        grid=(B,),
        in_specs=[
            pl.BlockSpec(memory_space=pltpu.MemorySpace.HBM),
            pl.BlockSpec(memory_space=pltpu.MemorySpace.HBM),
            pl.BlockSpec((1, 1, 4), lambda b: (b, 0, 0)),
        ],
        out_specs=[
            pl.BlockSpec((1, _KP, 1), lambda b: (b, 0, 0)),
            pl.BlockSpec((1, _KP, 1), lambda b: (b, 0, 0)),
            pl.BlockSpec((1, _KP, 4), lambda b: (b, 0, 0)),
        ],
        out_shape=[
            jax.ShapeDtypeStruct((B, _KP, 1), jnp.float32),
            jax.ShapeDtypeStruct((B, _KP, 1), jnp.int32),
            jax.ShapeDtypeStruct((B, _KP, 4), jnp.float32),
        ],
        scratch_shapes=[
            pltpu.VMEM((_M, _S), jnp.float32),
            pltpu.VMEM((_MR, _MC), jnp.float32),
            pltpu.VMEM((_MR, _MC), jnp.int32),
            pltpu.VMEM((_KP, 1), jnp.float32),
            pltpu.VMEM((_KP, 1), jnp.int32),
            pltpu.VMEM((_KP, 4), jnp.float32),
            pltpu.SemaphoreType.DMA,
            pltpu.SemaphoreType.DMA,
        ],
        compiler_params=pltpu.CompilerParams(
            dimension_semantics=("arbitrary",),
        ),
    )(xp, pred_boxes, scale)

    return (scores3[:, :_K, 0], labels3[:, :_K, 0], boxes3[:, :_K, :])


# speedup vs baseline: 9.5843x; 4.9763x over previous
import functools

import jax
import jax.numpy as jnp
from jax import lax
from jax.experimental import pallas as pl
from jax.experimental.pallas import tpu as pltpu
from jax.experimental.pallas import tpu_sc as plsc

_B, _N, _C = 16, 20000, 91
_NC = _N * _C
_S = 1000
_SP = 1024
_M = 1820
_MP = 1920
_G = 130
_NG = 14
_K = 100
_KP = 128
_BIG = 1 << 30
_NV = _SP // 16
_NCORE, _NSUB = 2, 16


def _pass1_body(x_ref, p_ref, c1h, o1h, c2h, o2h,
                cs1, os1, cs2, os2, sem):
    b = pl.program_id(0)
    for g in range(_NG):
        sl = pl.ds(g * _G, _G)
        blk = x_ref[0, sl, :]
        p = jax.nn.sigmoid(blk)
        p_ref[0, sl, pl.ds(0, _S)] = p
        p_ref[0, sl, pl.ds(_S, _SP - _S)] = jnp.full(
            (_G, _SP - _S), -2.0, jnp.float32)
        ii = lax.broadcasted_iota(jnp.int32, (_G, _S), 1)
        m1 = jnp.max(p, axis=1)
        a1 = jnp.min(jnp.where(p == m1[:, None], ii, _BIG), axis=1)
        p2 = jnp.where(ii == a1[:, None], jnp.float32(-2.0), p)
        m2 = jnp.max(p2, axis=1)
        a2 = jnp.min(jnp.where(p2 == m2[:, None], ii, _BIG), axis=1)
        cs1[sl] = m1
        os1[sl] = a1
        cs2[sl] = m2
        os2[sl] = a2
    tl = pl.ds(_M, _MP - _M)
    cs1[tl] = jnp.full((_MP - _M,), -1.0, jnp.float32)
    cs2[tl] = jnp.full((_MP - _M,), -1.0, jnp.float32)
    os1[tl] = jnp.zeros((_MP - _M,), jnp.int32)
    os2[tl] = jnp.zeros((_MP - _M,), jnp.int32)
    for src, dst in ((cs1, c1h), (os1, o1h), (cs2, c2h), (os2, o2h)):
        cp = pltpu.make_async_copy(src, dst.at[b], sem)
        cp.start()
        cp.wait()


def _pass1(x3):
    return pl.pallas_call(
        _pass1_body,
        grid=(_B,),
        in_specs=[pl.BlockSpec((1, _M, _S), lambda b: (b, 0, 0))],
        out_specs=[
            pl.BlockSpec((1, _M, _SP), lambda b: (b, 0, 0)),
            pl.BlockSpec(memory_space=pltpu.MemorySpace.HBM),
            pl.BlockSpec(memory_space=pltpu.MemorySpace.HBM),
            pl.BlockSpec(memory_space=pltpu.MemorySpace.HBM),
            pl.BlockSpec(memory_space=pltpu.MemorySpace.HBM),
        ],
        out_shape=[
            jax.ShapeDtypeStruct((_B, _M, _SP), jnp.float32),
            jax.ShapeDtypeStruct((_B, _MP), jnp.float32),
            jax.ShapeDtypeStruct((_B, _MP), jnp.int32),
            jax.ShapeDtypeStruct((_B, _MP), jnp.float32),
            jax.ShapeDtypeStruct((_B, _MP), jnp.int32),
        ],
        scratch_shapes=[
            pltpu.VMEM((_MP,), jnp.float32),
            pltpu.VMEM((_MP,), jnp.int32),
            pltpu.VMEM((_MP,), jnp.float32),
            pltpu.VMEM((_MP,), jnp.int32),
            pltpu.SemaphoreType.DMA,
        ],
        compiler_params=pltpu.CompilerParams(
            dimension_semantics=("arbitrary",),
        ),
    )(x3)


def _splat_f(x):
    return jnp.full((16,), x, jnp.float32)


def _splat_i(x):
    return jnp.full((16,), x, jnp.int32)


def _sc_body(c1h, o1h, c2h, o2h, ph, pbh, sch,
             scores_h, labels_h, boxes_h,
             c1, o1, c2, o2, sup, cb, boxflat, scalev,
             sc_v, gidx_v, bidx_v, lab_v, out_v, sem_box):
    w = lax.axis_index("s") * _NCORE + lax.axis_index("c")

    @pl.when(w < _B)
    def _work():
        b = w
        ila = lax.iota(jnp.int32, 16)
        lane0 = ila == 0

        boxcp = pltpu.make_async_copy(pbh.at[b], boxflat, sem_box)
        boxcp.start()
        pltpu.sync_copy(c1h.at[b], c1)
        pltpu.sync_copy(o1h.at[b], o1)
        pltpu.sync_copy(c2h.at[b], c2)
        pltpu.sync_copy(o2h.at[b], o2)
        pltpu.sync_copy(sch.at[b], scalev)

        for v in range(8):
            sup[pl.ds(v * 16, 16)] = _splat_f(-3.0)
        for g in range(_MP // 16):
            m = jnp.max(c1[pl.ds(g * 16, 16)])
            plsc.store_scatter(sup, [_splat_i(g)], _splat_f(m), mask=lane0)

        def body(k, carry):
            svs = [sup[pl.ds(v * 16, 16)] for v in range(8)]
            m = svs[0]
            for v in range(1, 8):
                m = jnp.maximum(m, svs[v])
            gm = jnp.max(m)
            cand = _splat_i(_BIG)
            for v in range(8):
                cand = jnp.minimum(
                    cand, jnp.where(svs[v] == gm, ila + v * 16, _BIG))
            g = jnp.min(cand)
            gvals = plsc.load_gather(c1, [g * 16 + ila])
            jc = g * 16 + jnp.min(jnp.where(gvals == gm, ila, _BIG))
            jv = _splat_i(jc)
            off = jnp.max(plsc.load_gather(o1, [jv]))
            gidx = jc * _S + off
            kv = _splat_i(k)
            plsc.store_scatter(sc_v, [kv], _splat_f(gm), mask=lane0)
            plsc.store_scatter(gidx_v, [kv], _splat_i(gidx), mask=lane0)
            v2 = jnp.max(plsc.load_gather(c2, [jv]))
            o2v = jnp.max(plsc.load_gather(o2, [jv]))

            def use_c2():
                plsc.store_scatter(c1, [jv], _splat_f(v2), mask=lane0)
                plsc.store_scatter(o1, [jv], _splat_i(o2v), mask=lane0)
                plsc.store_scatter(c2, [jv], _splat_f(-2.0), mask=lane0)

            def rescan():
                pltpu.sync_copy(ph.at[b, jc], cb)
                plsc.store_scatter(cb, [_splat_i(off)], _splat_f(-2.0),
                                   mask=lane0)

                for t in range(_KP // 16):
                    gv = gidx_v[pl.ds(t * 16, 16)]
                    jef = (gv.astype(jnp.float32) + 0.5) * \
                        jnp.float32(1.0 / _S)
                    jev = jef.astype(jnp.int32)
                    oev = gv - jev * _S
                    cnd_t = (jev == jv) & ((ila + t * 16) < kv)
                    idxv = jnp.where(cnd_t, oev, _SP - 8)
                    plsc.store_scatter(cb, [idxv], _splat_f(-2.0),
                                       mask=cnd_t)
                vm = _splat_f(-3.0)
                for v in range(_NV):
                    vm = jnp.maximum(vm, cb[pl.ds(v * 16, 16)])
                m1 = jnp.max(vm)
                cnd = _splat_i(_BIG)
                for v in range(_NV):
                    cnd = jnp.minimum(
                        cnd, jnp.where(cb[pl.ds(v * 16, 16)] == m1,
                                       ila + v * 16, _BIG))
                oo1 = jnp.min(cnd)
                plsc.store_scatter(cb, [_splat_i(oo1)], _splat_f(-2.0),
                                   mask=lane0)
                vm2 = _splat_f(-3.0)
                for v in range(_NV):
                    vm2 = jnp.maximum(vm2, cb[pl.ds(v * 16, 16)])
                m2 = jnp.max(vm2)
                cnd2 = _splat_i(_BIG)
                for v in range(_NV):
                    cnd2 = jnp.minimum(
                        cnd2, jnp.where(cb[pl.ds(v * 16, 16)] == m2,
                                        ila + v * 16, _BIG))
                oo2 = jnp.min(cnd2)
                plsc.store_scatter(c1, [jv], _splat_f(m1), mask=lane0)
                plsc.store_scatter(o1, [jv], _splat_i(oo1), mask=lane0)
                plsc.store_scatter(c2, [jv], _splat_f(m2), mask=lane0)
                plsc.store_scatter(o2, [jv], _splat_i(oo2), mask=lane0)

            lax.cond(v2 > -1.5, use_c2, rescan)
            nv = plsc.load_gather(c1, [g * 16 + ila])
            sm = jnp.max(nv)
            plsc.store_scatter(sup, [_splat_i(g)], _splat_f(sm), mask=lane0)
            return carry

        lax.fori_loop(0, _K, body, 0)

        for t in range(7):
            sl = pl.ds(t * 16, 16)
            gv = gidx_v[sl]
            bi = ((gv.astype(jnp.float32) + 0.5) *
                  jnp.float32(1.0 / _C)).astype(jnp.int32)
            bidx_v[sl] = bi
            lab_v[sl] = gv - bi * _C

        boxcp.wait()
        q = lax.shift_right_logical(ila, 2)
        r = ila & 3
        r1 = r & 1
        sgn = jnp.where(r >= 2, jnp.float32(0.5), jnp.float32(-0.5))
        scv = plsc.load_gather(scalev, [r])
        for kk in range(_K // 4):
            biv = plsc.load_gather(bidx_v, [kk * 4 + q])
            a = plsc.load_gather(boxflat, [biv * 4 + r1])
            bb = plsc.load_gather(boxflat, [biv * 4 + 2 + r1])
            out_v[pl.ds(kk * 16, 16)] = (a + sgn * bb) * scv

        pltpu.sync_copy(sc_v, scores_h.at[b])
        pltpu.sync_copy(lab_v, labels_h.at[b])
        pltpu.sync_copy(out_v, boxes_h.at[b])


def _sc_select(c1a, o1a, c2a, o2a, probs, pb2, scale128):
    mesh = plsc.VectorSubcoreMesh(core_axis_name="c", subcore_axis_name="s")
    fn = pl.kernel(
        _sc_body,
        out_type=[
            jax.ShapeDtypeStruct((_B, _KP), jnp.float32),
            jax.ShapeDtypeStruct((_B, _KP), jnp.int32),
            jax.ShapeDtypeStruct((_B, _KP * 4), jnp.float32),
        ],
        mesh=mesh,
        compiler_params=pltpu.CompilerParams(needs_layout_passes=False),
        scratch_types=[
            pltpu.VMEM((_MP,), jnp.float32),
            pltpu.VMEM((_MP,), jnp.int32),
            pltpu.VMEM((_MP,), jnp.float32),
            pltpu.VMEM((_MP,), jnp.int32),
            pltpu.VMEM((128,), jnp.float32),
            pltpu.VMEM((_SP,), jnp.float32),
            pltpu.VMEM((_N * 4,), jnp.float32),
            pltpu.VMEM((128,), jnp.float32),
            pltpu.VMEM((_KP,), jnp.float32),
            pltpu.VMEM((_KP,), jnp.int32),
            pltpu.VMEM((_KP,), jnp.int32),
            pltpu.VMEM((_KP,), jnp.int32),
            pltpu.VMEM((_KP * 4,), jnp.float32),
            pltpu.SemaphoreType.DMA,
        ],
    )
    return fn(c1a, o1a, c2a, o2a, probs, pb2, scale128)


@jax.jit
def kernel(pred_logits, pred_boxes, target_sizes):
    B = pred_logits.shape[0]
    x3 = pred_logits.reshape(B, _M, _S)
    pb2 = pred_boxes.reshape(B, _N * 4)
    ts = target_sizes.astype(jnp.float32)
    h = ts[:, 0:1]
    w = ts[:, 1:2]
    z = jnp.zeros((B, 124), jnp.float32)
    scale128 = jnp.concatenate([w, h, w, h, z], axis=1)

    probs, c1a, o1a, c2a, o2a = _pass1(x3)
    scores, labels, boxes = _sc_select(c1a, o1a, c2a, o2a, probs, pb2,
                                       scale128)
    return (scores[:, :_K], labels[:, :_K],
            boxes.reshape(B, _KP, 4)[:, :_K, :])
